# baseline (speedup 1.0000x reference)
import functools

import jax
import jax.numpy as jnp
from jax import lax
from jax.experimental import pallas as pl
from jax.experimental.pallas import tpu as pltpu
from jax.experimental.pallas import tpu_sc as plsc

N0_, N1_, N2_ = 10000, 5000, 2500
C_ = 128

N0P, N1P = 10240, 5120
BLK = 512
NC, NS = 2, 16
NW = NC * NS
KM = 96
NB = 3
NE = 6
ZC = 64


def _cdiv(a, b):
    return (a + b - 1) // b


def _pad_edges(src, dst, ew, n_nodes):
    e = src.shape[0]
    q = NW * KM * NE
    ep = q * _cdiv(e, q)
    pad = ep - e
    if pad:
        pidx = (jnp.arange(pad, dtype=jnp.int32) * 64) % n_nodes
        src = jnp.concatenate([src, pidx])
        dst = jnp.concatenate([dst, pidx])
        ew = jnp.concatenate([ew, jnp.zeros((pad,), ew.dtype)])
    return src, dst, ew, ep


def _deg_call(dst0, ew0, dst1, ew1, ep0, ep1):
    ch0 = ep0 // NW // KM
    ch1 = ep1 // NW // KM
    wps0 = N0P // NS
    wps1 = N1P // NS
    mesh = plsc.VectorSubcoreMesh(core_axis_name="c", subcore_axis_name="s")

    @functools.partial(
        pl.kernel,
        out_type=(
            jax.ShapeDtypeStruct((NC * N0P,), jnp.float32),
            jax.ShapeDtypeStruct((NC * N1P,), jnp.float32),
        ),
        mesh=mesh,
        scratch_types=[
            pltpu.VMEM((ch0, KM), jnp.int32),
            pltpu.VMEM((ch0, KM), jnp.float32),
            pltpu.VMEM((wps0,), jnp.float32),
            pltpu.VMEM_SHARED((N0P,), jnp.float32),
            pltpu.VMEM_SHARED((N1P,), jnp.float32),
            pltpu.SemaphoreType.DMA,
        ],
    )
    def deg_kernel(dst0_h, ew0_h, dst1_h, ew1_h, degp0_h, degp1_h,
                   dslab, wslab, dbuf, deg0_sh, deg1_sh, sem):
        c = lax.axis_index("c")
        s = lax.axis_index("s")
        w = c * NS + s
        for q in range(wps0 // 16):
            dbuf[pl.ds(q * 16, 16)] = jnp.zeros((16,), jnp.float32)
        pltpu.sync_copy(dbuf, deg0_sh.at[pl.ds(s * wps0, wps0)])
        pltpu.sync_copy(dbuf.at[pl.ds(0, wps1)],
                        deg1_sh.at[pl.ds(s * wps1, wps1)])
        plsc.subcore_barrier()

        for lvl, (ch, dh, deg_sh) in enumerate(
                ((ch0, dst0_h, deg0_sh), (ch1, dst1_h, deg1_sh))):
            eh = ew0_h if lvl == 0 else ew1_h
            pltpu.sync_copy(dh.at[w], dslab.at[pl.ds(0, ch)])
            pltpu.sync_copy(eh.at[w], wslab.at[pl.ds(0, ch)])
            descs = []
            for j in range(ch):
                if j >= 16:
                    descs[j - 16].wait()
                descs.append(pltpu.async_copy(
                    wslab.at[j], deg_sh.at[dslab.at[j]], sem, add=True))
            for d in descs[-16:] if ch >= 16 else descs:
                d.wait()

        plsc.subcore_barrier()
        pltpu.sync_copy(deg0_sh.at[pl.ds(s * wps0, wps0)], dbuf)
        pltpu.sync_copy(dbuf, degp0_h.at[pl.ds(c * N0P + s * wps0, wps0)])
        pltpu.sync_copy(deg1_sh.at[pl.ds(s * wps1, wps1)],
                        dbuf.at[pl.ds(0, wps1)])
        pltpu.sync_copy(dbuf.at[pl.ds(0, wps1)],
                        degp1_h.at[pl.ds(c * N1P + s * wps1, wps1)])

    return deg_kernel(dst0, ew0, dst1, ew1)


def _msg_call(hp, e3, n_pad, ep, stage):
    ch = ep // NW // KM
    assert ch % NE == 0 and ch >= 2 * NE
    rps = n_pad // NS
    nwo = rps // ZC
    mesh = plsc.VectorSubcoreMesh(core_axis_name="c", subcore_axis_name="s")

    @functools.partial(
        pl.kernel,
        out_type=jax.ShapeDtypeStruct((NC, n_pad, C_), jnp.float32),
        mesh=mesh,
        scratch_types=[
            [pltpu.VMEM((3, KM), jnp.int32)] * NE,
            [pltpu.VMEM((KM, C_), jnp.float32)] * NB,
            [pltpu.SemaphoreType.DMA] * NE,
            [pltpu.SemaphoreType.DMA] * NB,
            [pltpu.SemaphoreType.DMA] * NB,
            pltpu.VMEM_SHARED((n_pad, C_), jnp.float32),
        ] + ([pltpu.VMEM_SHARED((n_pad, C_), jnp.float32)] if stage else []),
    )
    def msg_kernel(hp_h, e3_h, out_h, eb, rb, si, sg, ss, acc, *tbl):
        table = tbl[0] if stage else None
        c = lax.axis_index("c")
        s = lax.axis_index("s")
        w = c * NS + s

        @pl.loop(0, KM)
        def _zrow(r):
            for q in range(C_ // 16):
                rb[0][r, pl.ds(q * 16, 16)] = jnp.zeros((16,), jnp.float32)

        zdescs = [
            pltpu.async_copy(rb[0].at[pl.ds(0, ZC)],
                             acc.at[pl.ds(s * rps + m * ZC, ZC)], sg[0])
            for m in range(nwo)
        ]
        if stage:
            for m in range(rps // ZC):
                sl = pl.ds(s * rps + m * ZC, ZC)
                par = 1 + m % 2
                pltpu.sync_copy(hp_h.at[sl], rb[par].at[pl.ds(0, ZC)])
                pltpu.sync_copy(rb[par].at[pl.ds(0, ZC)], table.at[sl])
        for d in zdescs:
            d.wait()
        plsc.subcore_barrier()

        gsrc = table if stage else hp_h
        cl = lambda j: jnp.minimum(j, ch - 1)

        def idxload(j, e):
            pltpu.async_copy(e3_h.at[w, cl(j)], eb[e], si[e])

        def idx_wait(j, e):
            pltpu.make_async_copy(e3_h.at[w, cl(j)], eb[e], si[e]).wait()

        def gather(b, e):
            pltpu.async_copy(gsrc.at[eb[e].at[0]], rb[b], sg[b])

        def gather_wait(b, e):
            pltpu.make_async_copy(gsrc.at[eb[e].at[0]], rb[b], sg[b]).wait()

        def scatter(b, e):
            pltpu.async_copy(rb[b], acc.at[eb[e].at[1]], ss[b], add=True)

        def scatter_wait(b, e):
            pltpu.make_async_copy(rb[b], acc.at[eb[e].at[1]], ss[b]).wait()

        def scale(b, e):
            @pl.loop(0, KM // 16)
            def _scale(g):
                wi = eb[e][2, pl.ds(g * 16, 16)]
                wv = wi.astype(jnp.float32) * (2.0 ** -24)
                for k in range(16):
                    r = g * 16 + k
                    wgt = jnp.broadcast_to(wv[k], (16,))
                    for q in range(C_ // 16):
                        sl = pl.ds(q * 16, 16)
                        rb[b][r, sl] = rb[b][r, sl] * wgt

        def slot(j, b, e, peel):
            gather_wait(b, e)
            if not peel or j >= 2:
                scatter_wait((b + 1) % NB, (e - 2) % NE)
            idxload(j + 3, (e + 3) % NE)
            idx_wait(j + 1, (e + 1) % NE)
            gather((b + 1) % NB, (e + 1) % NE)
            scale(b, e)
            scatter(b, e)

        idxload(0, 0)
        idxload(1, 1)
        idxload(2, 2)
        idx_wait(0, 0)
        gather(0, 0)
        for j in range(NE):
            slot(j, j % NB, j % NE, peel=True)

        @pl.loop(1, ch // NE)
        def _main(t):
            for i in range(NE):
                slot(t * NE + i, i % NB, i % NE, peel=False)

        gather_wait(ch % NB, ch % NE)
        idx_wait(ch, (ch + 1) % NE)
        idx_wait(ch, (ch + 2) % NE)
        for d in range(1, NB):
            scatter_wait((ch - d) % NB, (ch - d) % NE)

        plsc.subcore_barrier()

        odescs = []
        for m in range(nwo):
            par = m % 2
            sl = pl.ds(s * rps + m * ZC, ZC)
            if m >= 2:
                odescs[m - 2].wait()
            pltpu.sync_copy(acc.at[sl], rb[par].at[pl.ds(0, ZC)])
            odescs.append(pltpu.async_copy(rb[par].at[pl.ds(0, ZC)],
                                           out_h.at[c, sl], ss[par]))
        for d in odescs[-2:] if nwo >= 2 else odescs:
            d.wait()

    return msg_kernel(hp, e3)


def _dis(deg_a, deg_b):
    deg = deg_a + deg_b + 2.0
    return jnp.where(deg > 0, lax.rsqrt(jnp.maximum(deg, 1e-12)), 0.0)


def _tc1_call(x, xs_1, degp1, w0):
    nb = N1P // BLK
    nbx = _cdiv(N2_, BLK)

    def body(xp_ref, xs_ref, d0_ref, d1_ref, w_ref, hp_ref, dis_ref):
        i = pl.program_id(0)
        dis = _dis(d0_ref[0], d1_ref[0])
        dis_ref[...] = dis
        rows = i * BLK + lax.broadcasted_iota(jnp.int32, (BLK, 1), 0)
        flag = jnp.where(rows < N2_, 1.0, 0.0)
        xin = xs_ref[...] + flag * xp_ref[...]
        h = jnp.dot(xin, w_ref[...], preferred_element_type=jnp.float32)
        hp_ref[...] = h * dis

    return pl.pallas_call(
        body,
        grid=(nb,),
        in_specs=[
            pl.BlockSpec((BLK, C_), lambda i: (jnp.minimum(i, nbx - 1), 0)),
            pl.BlockSpec((BLK, C_), lambda i: (i, 0)),
            pl.BlockSpec((1, BLK, 1), lambda i: (0, i, 0)),
            pl.BlockSpec((1, BLK, 1), lambda i: (1, i, 0)),
            pl.BlockSpec((C_, C_), lambda i: (0, 0)),
        ],
        out_specs=[
            pl.BlockSpec((BLK, C_), lambda i: (i, 0)),
            pl.BlockSpec((BLK, 1), lambda i: (i, 0)),
        ],
        out_shape=[
            jax.ShapeDtypeStruct((N1P, C_), jnp.float32),
            jax.ShapeDtypeStruct((N1P, 1), jnp.float32),
        ],
    )(x, xs_1, degp1, degp1, w0)


def _tc2_call(accp1, hp1, dis1, b0, xs_0, degp0, w1):
    nb = N0P // BLK
    nb1 = N1P // BLK

    def body(a0_ref, a1_ref, hp1_ref, dis1_ref, b0_ref, xs_ref,
             d0_ref, d1_ref, w_ref, hp_ref, dis_ref):
        i = pl.program_id(0)
        t = dis1_ref[...] * (a0_ref[0] + a1_ref[0] + 2.0 * hp1_ref[...])
        t = jnp.maximum(t + b0_ref[...], 0.0)
        rows = i * BLK + lax.broadcasted_iota(jnp.int32, (BLK, 1), 0)
        t = jnp.where(rows < N1_, t, 0.0)
        flag = jnp.where(i < nb1, 1.0, 0.0)
        xin = xs_ref[...] + flag * t
        dis = _dis(d0_ref[0], d1_ref[0])
        dis_ref[...] = dis
        h = jnp.dot(xin, w_ref[...], preferred_element_type=jnp.float32)
        hp_ref[...] = h * dis

    lvl1 = lambda i: (jnp.minimum(i, nb1 - 1), 0)
    return pl.pallas_call(
        body,
        grid=(nb,),
        in_specs=[
            pl.BlockSpec((1, BLK, C_), lambda i: (0, jnp.minimum(i, nb1 - 1), 0)),
            pl.BlockSpec((1, BLK, C_), lambda i: (1, jnp.minimum(i, nb1 - 1), 0)),
            pl.BlockSpec((BLK, C_), lvl1),
            pl.BlockSpec((BLK, 1), lvl1),
            pl.BlockSpec((1, C_), lambda i: (0, 0)),
            pl.BlockSpec((BLK, C_), lambda i: (i, 0)),
            pl.BlockSpec((1, BLK, 1), lambda i: (0, i, 0)),
            pl.BlockSpec((1, BLK, 1), lambda i: (1, i, 0)),
            pl.BlockSpec((C_, C_), lambda i: (0, 0)),
        ],
        out_specs=[
            pl.BlockSpec((BLK, C_), lambda i: (i, 0)),
            pl.BlockSpec((BLK, 1), lambda i: (i, 0)),
        ],
        out_shape=[
            jax.ShapeDtypeStruct((N0_, C_), jnp.float32),
            jax.ShapeDtypeStruct((N0_, 1), jnp.float32),
        ],
    )(accp1, accp1, hp1, dis1, b0, xs_0, degp0, degp0, w1)


def _tc3_call(accp0, hp0, dis0, b1):
    blk = 400
    nb = N0_ // blk

    def body(a0_ref, a1_ref, hp_ref, dis_ref, b_ref, out_ref):
        out_ref[...] = (dis_ref[...]
                        * (a0_ref[0] + a1_ref[0] + 2.0 * hp_ref[...])
                        + b_ref[...])

    return pl.pallas_call(
        body,
        grid=(nb,),
        in_specs=[
            pl.BlockSpec((1, blk, C_), lambda i: (0, i, 0)),
            pl.BlockSpec((1, blk, C_), lambda i: (1, i, 0)),
            pl.BlockSpec((blk, C_), lambda i: (i, 0)),
            pl.BlockSpec((blk, 1), lambda i: (i, 0)),
            pl.BlockSpec((1, C_), lambda i: (0, 0)),
        ],
        out_specs=pl.BlockSpec((blk, C_), lambda i: (i, 0)),
        out_shape=jax.ShapeDtypeStruct((N0_, C_), jnp.float32),
    )(accp0, accp0, hp0, dis0, b1)


def kernel(x, xs_0, xs_1, edge_index_0, edge_index_1, edge_weight_0,
           edge_weight_1, perm_0, perm_1, W0, b0, W1, b1):
    del perm_0, perm_1

    src0, dst0, ew0, ep0 = _pad_edges(edge_index_0[0], edge_index_0[1],
                                      edge_weight_0, N0_)
    src1, dst1, ew1, ep1 = _pad_edges(edge_index_1[0], edge_index_1[1],
                                      edge_weight_1, N1_)

    def _e3(src, dst, ew, ep):
        ch = ep // NW // KM
        ewq = (ew * 16777216.0 + 0.5).astype(jnp.int32)
        return jnp.stack([a.reshape(NW, ch, KM) for a in (src, dst, ewq)],
                         axis=2)

    e30 = _e3(src0, dst0, ew0, ep0)
    e31 = _e3(src1, dst1, ew1, ep1)
    chd0 = ep0 // NW // KM
    chd1 = ep1 // NW // KM
    degp0, degp1 = _deg_call(dst0.reshape(NW, chd0, KM),
                             ew0.reshape(NW, chd0, KM),
                             dst1.reshape(NW, chd1, KM),
                             ew1.reshape(NW, chd1, KM), ep0, ep1)
    degp0 = degp0.reshape(NC, N0P, 1)
    degp1 = degp1.reshape(NC, N1P, 1)

    hp1, dis1 = _tc1_call(x, xs_1, degp1, W0)
    accp1 = _msg_call(hp1, e31, N1P, ep1, stage=True)
    hp0, dis0 = _tc2_call(accp1, hp1, dis1, b0.reshape(1, C_), xs_0,
                          degp0, W1)
    accp0 = _msg_call(hp0, e30, N0P, ep0, stage=False)
    return _tc3_call(accp0, hp0, dis0, b1.reshape(1, C_))


# speedup vs baseline: 1.0173x; 1.0173x over previous
import functools

import jax
import jax.numpy as jnp
from jax import lax
from jax.experimental import pallas as pl
from jax.experimental.pallas import tpu as pltpu
from jax.experimental.pallas import tpu_sc as plsc

N0_, N1_, N2_ = 10000, 5000, 2500
C_ = 128

N0P, N1P = 10240, 5120
BLK = 512
NC, NS = 2, 16
NW = NC * NS
KM = 96
NB = 3
NE = 6
ZC = 64


def _cdiv(a, b):
    return (a + b - 1) // b


def _pad_edges(src, dst, ew, n_nodes):
    e = src.shape[0]
    q = NW * KM * NE
    ep = q * _cdiv(e, q)
    pad = ep - e
    if pad:
        pidx = (jnp.arange(pad, dtype=jnp.int32) * 64) % n_nodes
        src = jnp.concatenate([src, pidx])
        dst = jnp.concatenate([dst, pidx])
        ew = jnp.concatenate([ew, jnp.zeros((pad,), ew.dtype)])
    return src, dst, ew, ep


def _deg_call(dst0, ew0, dst1, ew1, ep0, ep1):
    ch0 = ep0 // NW // KM
    ch1 = ep1 // NW // KM
    wps0 = N0P // NS
    wps1 = N1P // NS
    mesh = plsc.VectorSubcoreMesh(core_axis_name="c", subcore_axis_name="s")

    @functools.partial(
        pl.kernel,
        out_type=(
            jax.ShapeDtypeStruct((NC * N0P,), jnp.float32),
            jax.ShapeDtypeStruct((NC * N1P,), jnp.float32),
        ),
        mesh=mesh,
        scratch_types=[
            pltpu.VMEM((ch0, KM), jnp.int32),
            pltpu.VMEM((ch0, KM), jnp.float32),
            pltpu.VMEM((wps0,), jnp.float32),
            pltpu.VMEM_SHARED((N0P,), jnp.float32),
            pltpu.VMEM_SHARED((N1P,), jnp.float32),
            pltpu.SemaphoreType.DMA,
        ],
    )
    def deg_kernel(dst0_h, ew0_h, dst1_h, ew1_h, degp0_h, degp1_h,
                   dslab, wslab, dbuf, deg0_sh, deg1_sh, sem):
        c = lax.axis_index("c")
        s = lax.axis_index("s")
        w = c * NS + s
        for q in range(wps0 // 16):
            dbuf[pl.ds(q * 16, 16)] = jnp.zeros((16,), jnp.float32)
        pltpu.sync_copy(dbuf, deg0_sh.at[pl.ds(s * wps0, wps0)])
        pltpu.sync_copy(dbuf.at[pl.ds(0, wps1)],
                        deg1_sh.at[pl.ds(s * wps1, wps1)])
        plsc.subcore_barrier()

        for lvl, (ch, dh, deg_sh) in enumerate(
                ((ch0, dst0_h, deg0_sh), (ch1, dst1_h, deg1_sh))):
            eh = ew0_h if lvl == 0 else ew1_h
            pltpu.sync_copy(dh.at[w], dslab.at[pl.ds(0, ch)])
            pltpu.sync_copy(eh.at[w], wslab.at[pl.ds(0, ch)])
            descs = []
            for j in range(ch):
                if j >= 16:
                    descs[j - 16].wait()
                descs.append(pltpu.async_copy(
                    wslab.at[j], deg_sh.at[dslab.at[j]], sem, add=True))
            for d in descs[-16:] if ch >= 16 else descs:
                d.wait()

        plsc.subcore_barrier()
        pltpu.sync_copy(deg0_sh.at[pl.ds(s * wps0, wps0)], dbuf)
        pltpu.sync_copy(dbuf, degp0_h.at[pl.ds(c * N0P + s * wps0, wps0)])
        pltpu.sync_copy(deg1_sh.at[pl.ds(s * wps1, wps1)],
                        dbuf.at[pl.ds(0, wps1)])
        pltpu.sync_copy(dbuf.at[pl.ds(0, wps1)],
                        degp1_h.at[pl.ds(c * N1P + s * wps1, wps1)])

    return deg_kernel(dst0, ew0, dst1, ew1)


def _msg_call(hp, e3, n_pad, ep, stage):
    ch = ep // NW // KM
    assert ch % NE == 0 and ch >= 2 * NE
    rps = n_pad // NS
    nwo = rps // ZC
    mesh = plsc.VectorSubcoreMesh(core_axis_name="c", subcore_axis_name="s")

    @functools.partial(
        pl.kernel,
        out_type=jax.ShapeDtypeStruct((NC, n_pad, C_), jnp.float32),
        mesh=mesh,
        scratch_types=[
            [pltpu.VMEM((3, KM), jnp.int32)] * NE,
            [pltpu.VMEM((KM, C_), jnp.float32)] * NB,
            [pltpu.SemaphoreType.DMA] * NE,
            [pltpu.SemaphoreType.DMA] * NB,
            [pltpu.SemaphoreType.DMA] * NB,
            pltpu.VMEM_SHARED((n_pad, C_), jnp.float32),
        ] + ([pltpu.VMEM_SHARED((n_pad, C_), jnp.float32)] if stage else []),
    )
    def msg_kernel(hp_h, e3_h, out_h, eb, rb, si, sg, ss, acc, *tbl):
        table = tbl[0] if stage else None
        c = lax.axis_index("c")
        s = lax.axis_index("s")
        w = c * NS + s

        @pl.loop(0, KM)
        def _zrow(r):
            for q in range(C_ // 16):
                rb[0][r, pl.ds(q * 16, 16)] = jnp.zeros((16,), jnp.float32)

        zdescs = [
            pltpu.async_copy(rb[0].at[pl.ds(0, ZC)],
                             acc.at[pl.ds(s * rps + m * ZC, ZC)], sg[0])
            for m in range(nwo)
        ]
        if stage:
            for m in range(rps // ZC):
                sl = pl.ds(s * rps + m * ZC, ZC)
                par = 1 + m % 2
                pltpu.sync_copy(hp_h.at[sl], rb[par].at[pl.ds(0, ZC)])
                pltpu.sync_copy(rb[par].at[pl.ds(0, ZC)], table.at[sl])
        for d in zdescs:
            d.wait()
        plsc.subcore_barrier()

        gsrc = table if stage else hp_h
        cl = lambda j: jnp.minimum(j, ch - 1)

        def idxload(j, e):
            pltpu.async_copy(e3_h.at[w, cl(j)], eb[e], si[e])

        def idx_wait(j, e):
            pltpu.make_async_copy(e3_h.at[w, cl(j)], eb[e], si[e]).wait()

        def gather(b, e):
            pltpu.async_copy(gsrc.at[eb[e].at[0]], rb[b], sg[b])

        def gather_wait(b, e):
            pltpu.make_async_copy(gsrc.at[eb[e].at[0]], rb[b], sg[b]).wait()

        def scatter(b, e):
            pltpu.async_copy(rb[b], acc.at[eb[e].at[1]], ss[b], add=True)

        def scatter_wait(b, e):
            pltpu.make_async_copy(rb[b], acc.at[eb[e].at[1]], ss[b]).wait()

        def scale(b, e):
            @pl.loop(0, KM // 16)
            def _scale(g):
                wi = eb[e][2, pl.ds(g * 16, 16)]
                wv = wi.astype(jnp.float32) * (2.0 ** -24)
                for k in range(16):
                    r = g * 16 + k
                    wgt = jnp.broadcast_to(wv[k], (16,))
                    for q in range(C_ // 16):
                        sl = pl.ds(q * 16, 16)
                        rb[b][r, sl] = rb[b][r, sl] * wgt

        def slot(j, b, e, peel):
            gather_wait(b, e)
            if not peel or j >= 2:
                scatter_wait((b + 1) % NB, (e - 2) % NE)
            idxload(j + 3, (e + 3) % NE)
            idx_wait(j + 1, (e + 1) % NE)
            gather((b + 1) % NB, (e + 1) % NE)
            scale(b, e)
            scatter(b, e)

        idxload(0, 0)
        idxload(1, 1)
        idxload(2, 2)
        idx_wait(0, 0)
        gather(0, 0)
        for j in range(NE):
            slot(j, j % NB, j % NE, peel=True)

        @pl.loop(1, ch // NE)
        def _main(t):
            for i in range(NE):
                slot(t * NE + i, i % NB, i % NE, peel=False)

        gather_wait(ch % NB, ch % NE)
        idx_wait(ch, (ch + 1) % NE)
        idx_wait(ch, (ch + 2) % NE)
        for d in range(1, NB):
            scatter_wait((ch - d) % NB, (ch - d) % NE)

        plsc.subcore_barrier()

        odescs = []
        for m in range(nwo):
            par = m % 2
            sl = pl.ds(s * rps + m * ZC, ZC)
            if m >= 2:
                odescs[m - 2].wait()
            pltpu.sync_copy(acc.at[sl], rb[par].at[pl.ds(0, ZC)])
            odescs.append(pltpu.async_copy(rb[par].at[pl.ds(0, ZC)],
                                           out_h.at[c, sl], ss[par]))
        for d in odescs[-2:] if nwo >= 2 else odescs:
            d.wait()

    return msg_kernel(hp, e3)


def _dis(deg_a, deg_b):
    deg = deg_a + deg_b + 2.0
    return jnp.where(deg > 0, lax.rsqrt(jnp.maximum(deg, 1e-12)), 0.0)


def _tc1_call(x, xs_1, degp1, w0):
    nb = N1P // BLK
    nbx = _cdiv(N2_, BLK)

    def body(xp_ref, xs_ref, d0_ref, d1_ref, w_ref, hp_ref, dis_ref):
        i = pl.program_id(0)
        dis = _dis(d0_ref[0], d1_ref[0])
        dis_ref[...] = dis
        rows = i * BLK + lax.broadcasted_iota(jnp.int32, (BLK, 1), 0)
        flag = jnp.where(rows < N2_, 1.0, 0.0)
        xin = xs_ref[...] + flag * xp_ref[...]
        h = jnp.dot(xin, w_ref[...], preferred_element_type=jnp.float32)
        hp_ref[...] = h * dis

    return pl.pallas_call(
        body,
        grid=(nb,),
        in_specs=[
            pl.BlockSpec((BLK, C_), lambda i: (jnp.minimum(i, nbx - 1), 0)),
            pl.BlockSpec((BLK, C_), lambda i: (i, 0)),
            pl.BlockSpec((1, BLK, 1), lambda i: (0, i, 0)),
            pl.BlockSpec((1, BLK, 1), lambda i: (1, i, 0)),
            pl.BlockSpec((C_, C_), lambda i: (0, 0)),
        ],
        out_specs=[
            pl.BlockSpec((BLK, C_), lambda i: (i, 0)),
            pl.BlockSpec((BLK, 1), lambda i: (i, 0)),
        ],
        out_shape=[
            jax.ShapeDtypeStruct((N1P, C_), jnp.float32),
            jax.ShapeDtypeStruct((N1P, 1), jnp.float32),
        ],
    )(x, xs_1, degp1, degp1, w0)


def _tc2_call(accp1, hp1, dis1, b0, xs_0, degp0, w1):
    nb = N0P // BLK
    nb1 = N1P // BLK

    def body(a0_ref, a1_ref, hp1_ref, dis1_ref, b0_ref, xs_ref,
             d0_ref, d1_ref, w_ref, hp_ref, dis_ref):
        i = pl.program_id(0)
        t = dis1_ref[...] * (a0_ref[0] + a1_ref[0] + 2.0 * hp1_ref[...])
        t = jnp.maximum(t + b0_ref[...], 0.0)
        rows = i * BLK + lax.broadcasted_iota(jnp.int32, (BLK, 1), 0)
        t = jnp.where(rows < N1_, t, 0.0)
        flag = jnp.where(i < nb1, 1.0, 0.0)
        xin = xs_ref[...] + flag * t
        dis = _dis(d0_ref[0], d1_ref[0])
        dis_ref[...] = dis
        h = jnp.dot(xin, w_ref[...], preferred_element_type=jnp.float32)
        hp_ref[...] = h * dis

    lvl1 = lambda i: (jnp.minimum(i, nb1 - 1), 0)
    return pl.pallas_call(
        body,
        grid=(nb,),
        in_specs=[
            pl.BlockSpec((1, BLK, C_), lambda i: (0, jnp.minimum(i, nb1 - 1), 0)),
            pl.BlockSpec((1, BLK, C_), lambda i: (1, jnp.minimum(i, nb1 - 1), 0)),
            pl.BlockSpec((BLK, C_), lvl1),
            pl.BlockSpec((BLK, 1), lvl1),
            pl.BlockSpec((1, C_), lambda i: (0, 0)),
            pl.BlockSpec((BLK, C_), lambda i: (i, 0)),
            pl.BlockSpec((1, BLK, 1), lambda i: (0, i, 0)),
            pl.BlockSpec((1, BLK, 1), lambda i: (1, i, 0)),
            pl.BlockSpec((C_, C_), lambda i: (0, 0)),
        ],
        out_specs=[
            pl.BlockSpec((BLK, C_), lambda i: (i, 0)),
            pl.BlockSpec((BLK, 1), lambda i: (i, 0)),
        ],
        out_shape=[
            jax.ShapeDtypeStruct((N0_, C_), jnp.float32),
            jax.ShapeDtypeStruct((N0_, 1), jnp.float32),
        ],
    )(accp1, accp1, hp1, dis1, b0, xs_0, degp0, degp0, w1)


def _tc3_call(accp0, hp0, dis0, b1):
    blk = 400
    nb = N0_ // blk

    def body(a0_ref, a1_ref, hp_ref, dis_ref, b_ref, out_ref):
        out_ref[...] = (dis_ref[...]
                        * (a0_ref[0] + a1_ref[0] + 2.0 * hp_ref[...])
                        + b_ref[...])

    return pl.pallas_call(
        body,
        grid=(nb,),
        in_specs=[
            pl.BlockSpec((1, blk, C_), lambda i: (0, i, 0)),
            pl.BlockSpec((1, blk, C_), lambda i: (1, i, 0)),
            pl.BlockSpec((blk, C_), lambda i: (i, 0)),
            pl.BlockSpec((blk, 1), lambda i: (i, 0)),
            pl.BlockSpec((1, C_), lambda i: (0, 0)),
        ],
        out_specs=pl.BlockSpec((blk, C_), lambda i: (i, 0)),
        out_shape=jax.ShapeDtypeStruct((N0_, C_), jnp.float32),
    )(accp0, accp0, hp0, dis0, b1)


def kernel(x, xs_0, xs_1, edge_index_0, edge_index_1, edge_weight_0,
           edge_weight_1, perm_0, perm_1, W0, b0, W1, b1):
    del perm_0, perm_1

    src0, dst0, ew0, ep0 = _pad_edges(edge_index_0[0], edge_index_0[1],
                                      edge_weight_0, N0_)
    src1, dst1, ew1, ep1 = _pad_edges(edge_index_1[0], edge_index_1[1],
                                      edge_weight_1, N1_)

    def _e3(src, dst, ew, ep):
        ch = ep // NW // KM
        ewq = (ew * 16777216.0 + 0.5).astype(jnp.int32)
        return jnp.stack([a.reshape(NW, ch, KM) for a in (src, dst, ewq)],
                         axis=2)

    e30 = _e3(src0, dst0, ew0, ep0)
    e31 = _e3(src1, dst1, ew1, ep1)
    chd0 = ep0 // NW // KM
    chd1 = ep1 // NW // KM
    degp0, degp1 = _deg_call(dst0.reshape(NW, chd0, KM),
                             ew0.reshape(NW, chd0, KM),
                             dst1.reshape(NW, chd1, KM),
                             ew1.reshape(NW, chd1, KM), ep0, ep1)
    degp0 = degp0.reshape(NC, N0P, 1)
    degp1 = degp1.reshape(NC, N1P, 1)

    hp1, dis1 = _tc1_call(x, xs_1, degp1, W0)
    accp1 = _msg_call(hp1, e31, N1P, ep1, stage=False)
    hp0, dis0 = _tc2_call(accp1, hp1, dis1, b0.reshape(1, C_), xs_0,
                          degp0, W1)
    accp0 = _msg_call(hp0, e30, N0P, ep0, stage=False)
    return _tc3_call(accp0, hp0, dis0, b1.reshape(1, C_))


# speedup vs baseline: 1.0424x; 1.0247x over previous
import functools

import jax
import jax.numpy as jnp
from jax import lax
from jax.experimental import pallas as pl
from jax.experimental.pallas import tpu as pltpu
from jax.experimental.pallas import tpu_sc as plsc

N0_, N1_, N2_ = 10000, 5000, 2500
C_ = 128

N0P, N1P = 10240, 5120
BLK = 512
NC, NS = 2, 16
NW = NC * NS
KM = 112
NB = 3
NE = 6
ZC = 64


def _cdiv(a, b):
    return (a + b - 1) // b


def _pad_edges(src, dst, ew, n_nodes):
    e = src.shape[0]
    q = NW * KM * NE
    ep = q * _cdiv(e, q)
    pad = ep - e
    if pad:
        pidx = (jnp.arange(pad, dtype=jnp.int32) * 64) % n_nodes
        src = jnp.concatenate([src, pidx])
        dst = jnp.concatenate([dst, pidx])
        ew = jnp.concatenate([ew, jnp.zeros((pad,), ew.dtype)])
    return src, dst, ew, ep


def _deg_call(dst0, ew0, dst1, ew1, ep0, ep1):
    ch0 = ep0 // NW // KM
    ch1 = ep1 // NW // KM
    wps0 = N0P // NS
    wps1 = N1P // NS
    mesh = plsc.VectorSubcoreMesh(core_axis_name="c", subcore_axis_name="s")

    @functools.partial(
        pl.kernel,
        out_type=(
            jax.ShapeDtypeStruct((NC * N0P,), jnp.float32),
            jax.ShapeDtypeStruct((NC * N1P,), jnp.float32),
        ),
        mesh=mesh,
        scratch_types=[
            pltpu.VMEM((ch0, KM), jnp.int32),
            pltpu.VMEM((ch0, KM), jnp.float32),
            pltpu.VMEM((wps0,), jnp.float32),
            pltpu.VMEM_SHARED((N0P,), jnp.float32),
            pltpu.VMEM_SHARED((N1P,), jnp.float32),
            pltpu.SemaphoreType.DMA,
        ],
    )
    def deg_kernel(dst0_h, ew0_h, dst1_h, ew1_h, degp0_h, degp1_h,
                   dslab, wslab, dbuf, deg0_sh, deg1_sh, sem):
        c = lax.axis_index("c")
        s = lax.axis_index("s")
        w = c * NS + s
        for q in range(wps0 // 16):
            dbuf[pl.ds(q * 16, 16)] = jnp.zeros((16,), jnp.float32)
        pltpu.sync_copy(dbuf, deg0_sh.at[pl.ds(s * wps0, wps0)])
        pltpu.sync_copy(dbuf.at[pl.ds(0, wps1)],
                        deg1_sh.at[pl.ds(s * wps1, wps1)])
        plsc.subcore_barrier()

        for lvl, (ch, dh, deg_sh) in enumerate(
                ((ch0, dst0_h, deg0_sh), (ch1, dst1_h, deg1_sh))):
            eh = ew0_h if lvl == 0 else ew1_h
            pltpu.sync_copy(dh.at[w], dslab.at[pl.ds(0, ch)])
            pltpu.sync_copy(eh.at[w], wslab.at[pl.ds(0, ch)])
            descs = []
            for j in range(ch):
                if j >= 16:
                    descs[j - 16].wait()
                descs.append(pltpu.async_copy(
                    wslab.at[j], deg_sh.at[dslab.at[j]], sem, add=True))
            for d in descs[-16:] if ch >= 16 else descs:
                d.wait()

        plsc.subcore_barrier()
        pltpu.sync_copy(deg0_sh.at[pl.ds(s * wps0, wps0)], dbuf)
        pltpu.sync_copy(dbuf, degp0_h.at[pl.ds(c * N0P + s * wps0, wps0)])
        pltpu.sync_copy(deg1_sh.at[pl.ds(s * wps1, wps1)],
                        dbuf.at[pl.ds(0, wps1)])
        pltpu.sync_copy(dbuf.at[pl.ds(0, wps1)],
                        degp1_h.at[pl.ds(c * N1P + s * wps1, wps1)])

    return deg_kernel(dst0, ew0, dst1, ew1)


def _msg_call(hp, e3, n_pad, ep):
    ch = ep // NW // KM
    assert ch % NE == 0 and ch >= 2 * NE
    rps = n_pad // NS
    nwo = rps // ZC
    mesh = plsc.VectorSubcoreMesh(core_axis_name="c", subcore_axis_name="s")

    @functools.partial(
        pl.kernel,
        out_type=jax.ShapeDtypeStruct((NC, n_pad, C_), jnp.float32),
        mesh=mesh,
        scratch_types=[
            [pltpu.VMEM((3, KM), jnp.int32)] * NE,
            [pltpu.VMEM((KM, C_), jnp.float32)] * NB,
            [pltpu.SemaphoreType.DMA] * NE,
            [pltpu.SemaphoreType.DMA] * NB,
            [pltpu.SemaphoreType.DMA] * NB,
            pltpu.VMEM_SHARED((n_pad, C_), jnp.float32),
        ],
    )
    def msg_kernel(hp_h, e3_h, out_h, eb, rb, si, sg, ss, acc):
        c = lax.axis_index("c")
        s = lax.axis_index("s")
        w = c * NS + s

        @pl.loop(0, KM)
        def _zrow(r):
            for q in range(C_ // 16):
                rb[0][r, pl.ds(q * 16, 16)] = jnp.zeros((16,), jnp.float32)

        zdescs = [
            pltpu.async_copy(rb[0].at[pl.ds(0, ZC)],
                             acc.at[pl.ds(s * rps + m * ZC, ZC)], sg[0])
            for m in range(nwo)
        ]
        for d in zdescs:
            d.wait()
        plsc.subcore_barrier()

        gsrc = hp_h
        cl = lambda j: jnp.minimum(j, ch - 1)

        def idxload(j, e):
            pltpu.async_copy(e3_h.at[w, cl(j)], eb[e], si[e])

        def idx_wait(j, e):
            pltpu.make_async_copy(e3_h.at[w, cl(j)], eb[e], si[e]).wait()

        def gather(b, e):
            pltpu.async_copy(gsrc.at[eb[e].at[0]], rb[b], sg[b])

        def gather_wait(b, e):
            pltpu.make_async_copy(gsrc.at[eb[e].at[0]], rb[b], sg[b]).wait()

        def scatter(b, e):
            pltpu.async_copy(rb[b], acc.at[eb[e].at[1]], ss[b], add=True)

        def scatter_wait(b, e):
            pltpu.make_async_copy(rb[b], acc.at[eb[e].at[1]], ss[b]).wait()

        def scale(b, e):
            @pl.loop(0, KM // 16)
            def _scale(g):
                wi = eb[e][2, pl.ds(g * 16, 16)]
                wv = wi.astype(jnp.float32) * (2.0 ** -24)
                for k in range(16):
                    r = g * 16 + k
                    wgt = jnp.broadcast_to(wv[k], (16,))
                    for q in range(C_ // 16):
                        sl = pl.ds(q * 16, 16)
                        rb[b][r, sl] = rb[b][r, sl] * wgt

        def slot(j, b, e, peel):
            gather_wait(b, e)
            if not peel or j >= 2:
                scatter_wait((b + 1) % NB, (e - 2) % NE)
            idxload(j + 3, (e + 3) % NE)
            idx_wait(j + 1, (e + 1) % NE)
            gather((b + 1) % NB, (e + 1) % NE)
            scale(b, e)
            scatter(b, e)

        idxload(0, 0)
        idxload(1, 1)
        idxload(2, 2)
        idx_wait(0, 0)
        gather(0, 0)
        for j in range(NE):
            slot(j, j % NB, j % NE, peel=True)

        @pl.loop(1, ch // NE)
        def _main(t):
            for i in range(NE):
                slot(t * NE + i, i % NB, i % NE, peel=False)

        gather_wait(ch % NB, ch % NE)
        idx_wait(ch, (ch + 1) % NE)
        idx_wait(ch, (ch + 2) % NE)
        for d in range(1, NB):
            scatter_wait((ch - d) % NB, (ch - d) % NE)

        plsc.subcore_barrier()

        odescs = []
        for m in range(nwo):
            par = m % 2
            sl = pl.ds(s * rps + m * ZC, ZC)
            if m >= 2:
                odescs[m - 2].wait()
            pltpu.sync_copy(acc.at[sl], rb[par].at[pl.ds(0, ZC)])
            odescs.append(pltpu.async_copy(rb[par].at[pl.ds(0, ZC)],
                                           out_h.at[c, sl], ss[par]))
        for d in odescs[-2:] if nwo >= 2 else odescs:
            d.wait()

    return msg_kernel(hp, e3)


def _dis(deg_a, deg_b):
    deg = deg_a + deg_b + 2.0
    return jnp.where(deg > 0, lax.rsqrt(jnp.maximum(deg, 1e-12)), 0.0)


def _tc1_call(x, xs_1, degp1, w0):
    nb = N1P // BLK
    nbx = _cdiv(N2_, BLK)

    def body(xp_ref, xs_ref, d0_ref, d1_ref, w_ref, hp_ref, dis_ref):
        i = pl.program_id(0)
        dis = _dis(d0_ref[0], d1_ref[0])
        dis_ref[...] = dis
        rows = i * BLK + lax.broadcasted_iota(jnp.int32, (BLK, 1), 0)
        flag = jnp.where(rows < N2_, 1.0, 0.0)
        xin = xs_ref[...] + flag * xp_ref[...]
        h = jnp.dot(xin, w_ref[...], preferred_element_type=jnp.float32)
        hp_ref[...] = h * dis

    return pl.pallas_call(
        body,
        grid=(nb,),
        in_specs=[
            pl.BlockSpec((BLK, C_), lambda i: (jnp.minimum(i, nbx - 1), 0)),
            pl.BlockSpec((BLK, C_), lambda i: (i, 0)),
            pl.BlockSpec((1, BLK, 1), lambda i: (0, i, 0)),
            pl.BlockSpec((1, BLK, 1), lambda i: (1, i, 0)),
            pl.BlockSpec((C_, C_), lambda i: (0, 0)),
        ],
        out_specs=[
            pl.BlockSpec((BLK, C_), lambda i: (i, 0)),
            pl.BlockSpec((BLK, 1), lambda i: (i, 0)),
        ],
        out_shape=[
            jax.ShapeDtypeStruct((N1P, C_), jnp.float32),
            jax.ShapeDtypeStruct((N1P, 1), jnp.float32),
        ],
    )(x, xs_1, degp1, degp1, w0)


def _tc2_call(accp1, hp1, dis1, b0, xs_0, degp0, w1):
    nb = N0P // BLK
    nb1 = N1P // BLK

    def body(a0_ref, a1_ref, hp1_ref, dis1_ref, b0_ref, xs_ref,
             d0_ref, d1_ref, w_ref, hp_ref, dis_ref):
        i = pl.program_id(0)
        t = dis1_ref[...] * (a0_ref[0] + a1_ref[0] + 2.0 * hp1_ref[...])
        t = jnp.maximum(t + b0_ref[...], 0.0)
        rows = i * BLK + lax.broadcasted_iota(jnp.int32, (BLK, 1), 0)
        t = jnp.where(rows < N1_, t, 0.0)
        flag = jnp.where(i < nb1, 1.0, 0.0)
        xin = xs_ref[...] + flag * t
        dis = _dis(d0_ref[0], d1_ref[0])
        dis_ref[...] = dis
        h = jnp.dot(xin, w_ref[...], preferred_element_type=jnp.float32)
        hp_ref[...] = h * dis

    lvl1 = lambda i: (jnp.minimum(i, nb1 - 1), 0)
    return pl.pallas_call(
        body,
        grid=(nb,),
        in_specs=[
            pl.BlockSpec((1, BLK, C_), lambda i: (0, jnp.minimum(i, nb1 - 1), 0)),
            pl.BlockSpec((1, BLK, C_), lambda i: (1, jnp.minimum(i, nb1 - 1), 0)),
            pl.BlockSpec((BLK, C_), lvl1),
            pl.BlockSpec((BLK, 1), lvl1),
            pl.BlockSpec((1, C_), lambda i: (0, 0)),
            pl.BlockSpec((BLK, C_), lambda i: (i, 0)),
            pl.BlockSpec((1, BLK, 1), lambda i: (0, i, 0)),
            pl.BlockSpec((1, BLK, 1), lambda i: (1, i, 0)),
            pl.BlockSpec((C_, C_), lambda i: (0, 0)),
        ],
        out_specs=[
            pl.BlockSpec((BLK, C_), lambda i: (i, 0)),
            pl.BlockSpec((BLK, 1), lambda i: (i, 0)),
        ],
        out_shape=[
            jax.ShapeDtypeStruct((N0_, C_), jnp.float32),
            jax.ShapeDtypeStruct((N0_, 1), jnp.float32),
        ],
    )(accp1, accp1, hp1, dis1, b0, xs_0, degp0, degp0, w1)


def _tc3_call(accp0, hp0, dis0, b1):
    blk = 400
    nb = N0_ // blk

    def body(a0_ref, a1_ref, hp_ref, dis_ref, b_ref, out_ref):
        out_ref[...] = (dis_ref[...]
                        * (a0_ref[0] + a1_ref[0] + 2.0 * hp_ref[...])
                        + b_ref[...])

    return pl.pallas_call(
        body,
        grid=(nb,),
        in_specs=[
            pl.BlockSpec((1, blk, C_), lambda i: (0, i, 0)),
            pl.BlockSpec((1, blk, C_), lambda i: (1, i, 0)),
            pl.BlockSpec((blk, C_), lambda i: (i, 0)),
            pl.BlockSpec((blk, 1), lambda i: (i, 0)),
            pl.BlockSpec((1, C_), lambda i: (0, 0)),
        ],
        out_specs=pl.BlockSpec((blk, C_), lambda i: (i, 0)),
        out_shape=jax.ShapeDtypeStruct((N0_, C_), jnp.float32),
    )(accp0, accp0, hp0, dis0, b1)


def kernel(x, xs_0, xs_1, edge_index_0, edge_index_1, edge_weight_0,
           edge_weight_1, perm_0, perm_1, W0, b0, W1, b1):
    del perm_0, perm_1

    src0, dst0, ew0, ep0 = _pad_edges(edge_index_0[0], edge_index_0[1],
                                      edge_weight_0, N0_)
    src1, dst1, ew1, ep1 = _pad_edges(edge_index_1[0], edge_index_1[1],
                                      edge_weight_1, N1_)

    def _e3(src, dst, ew, ep):
        ch = ep // NW // KM
        ewq = (ew * 16777216.0 + 0.5).astype(jnp.int32)
        return jnp.stack([a.reshape(NW, ch, KM) for a in (src, dst, ewq)],
                         axis=2)

    e30 = _e3(src0, dst0, ew0, ep0)
    e31 = _e3(src1, dst1, ew1, ep1)
    chd0 = ep0 // NW // KM
    chd1 = ep1 // NW // KM
    degp0, degp1 = _deg_call(dst0.reshape(NW, chd0, KM),
                             ew0.reshape(NW, chd0, KM),
                             dst1.reshape(NW, chd1, KM),
                             ew1.reshape(NW, chd1, KM), ep0, ep1)
    degp0 = degp0.reshape(NC, N0P, 1)
    degp1 = degp1.reshape(NC, N1P, 1)

    hp1, dis1 = _tc1_call(x, xs_1, degp1, W0)
    accp1 = _msg_call(hp1, e31, N1P, ep1)
    hp0, dis0 = _tc2_call(accp1, hp1, dis1, b0.reshape(1, C_), xs_0,
                          degp0, W1)
    accp0 = _msg_call(hp0, e30, N0P, ep0)
    return _tc3_call(accp0, hp0, dis0, b1.reshape(1, C_))
